# baseline (speedup 1.0000x reference)
import functools

import jax
import jax.numpy as jnp
from jax.experimental import pallas as pl
from jax.experimental.pallas import tpu as pltpu

_NPOINT = 1024
_NSAMPLE = 32
_TILE = 128


def _fps_body(xt_ref, nx_ref):
    _, R, H = xt_ref.shape
    B = R // 2
    xs = xt_ref[0]
    ys = xt_ref[1]
    zs = xt_ref[2]
    il = jax.lax.broadcasted_iota(jnp.int32, (R, H), 1).astype(jnp.float32)
    rh = jax.lax.broadcasted_iota(jnp.int32, (R, H), 0)
    lanes = il + jnp.where(rh >= B, jnp.float32(H), jnp.float32(0.0))
    cols = jax.lax.broadcasted_iota(jnp.int32, (B, _NPOINT), 1)
    nbig = jnp.float32(2 * H)

    def _pair(v, op):
        lo = jax.lax.slice(v, (0, 0), (B, 1))
        hi = jax.lax.slice(v, (B, 0), (R, 1))
        four = op(lo, hi)
        return four, jnp.concatenate([four, four], axis=0)

    def body(i, carry):
        dists, far, nxx, nxy, nxz = carry
        mask = lanes == far
        cx4, cx = _pair(jnp.sum(jnp.where(mask, xs, 0.0), axis=1,
                                keepdims=True), jnp.add)
        cy4, cy = _pair(jnp.sum(jnp.where(mask, ys, 0.0), axis=1,
                                keepdims=True), jnp.add)
        cz4, cz = _pair(jnp.sum(jnp.where(mask, zs, 0.0), axis=1,
                                keepdims=True), jnp.add)
        here = cols == i
        nxx = jnp.where(here, cx4, nxx)
        nxy = jnp.where(here, cy4, nxy)
        nxz = jnp.where(here, cz4, nxz)
        dx = xs - cx
        dy = ys - cy
        dz = zs - cz
        d = (dx * dx + dy * dy) + dz * dz
        dists = jnp.minimum(dists, d)
        _, m = _pair(jnp.max(dists, axis=1, keepdims=True), jnp.maximum)
        _, far = _pair(jnp.min(jnp.where(dists == m, lanes, nbig),
                               axis=1, keepdims=True), jnp.minimum)
        return dists, far, nxx, nxy, nxz

    dists0 = jnp.full((R, H), 1e10, dtype=jnp.float32)
    far0 = jnp.zeros((R, 1), dtype=jnp.float32)
    nx0 = jnp.zeros((B, _NPOINT), dtype=jnp.float32)
    _, _, nxx, nxy, nxz = jax.lax.fori_loop(
        0, _NPOINT, body, (dists0, far0, nx0, nx0, nx0))
    nx_ref[0] = nxx
    nx_ref[1] = nxy
    nx_ref[2] = nxz


def _group_mlp_body(tbl_ref, nx_ref, w1_ref, b1_ref, w2_ref, b2_ref,
                    ws_ref, bs_ref, out_ref, d2_ref):
    _, CIN, N = tbl_ref.shape
    tbl = tbl_ref[0]
    xt = tbl[0:3, :]
    c = nx_ref[0]

    xsq = xt[0:1] ** 2 + xt[1:2] ** 2 + xt[2:3] ** 2
    csq = jnp.sum(c * c, axis=1, keepdims=True)
    cross = jax.lax.dot_general(c, xt, (((1,), (0,)), ((), ())),
                                preferred_element_type=jnp.float32)
    d2_ref[...] = csq + xsq - 2.0 * cross

    w1 = w1_ref[...]
    cw1 = jax.lax.dot_general(c, w1[0:3, :], (((1,), (0,)), ((), ())),
                              preferred_element_type=jnp.float32)
    b1 = b1_ref[...]
    w2 = w2_ref[...]
    b2 = b2_ref[...]
    lanes = jax.lax.broadcasted_iota(jnp.int32, (_TILE, N), 1).astype(jnp.float32)
    big = jnp.float32(3.0e38)

    def body(j, pooled):
        d2c = d2_ref[...]
        m = jnp.min(d2c, axis=1, keepdims=True)
        idx = jnp.min(jnp.where(d2c == m, lanes, jnp.float32(N)),
                      axis=1, keepdims=True)
        sel = lanes == idx
        d2_ref[...] = jnp.where(sel, big, d2c)
        onehot = sel.astype(jnp.float32)
        gath = jax.lax.dot_general(
            onehot, tbl, (((1,), (1,)), ((), ())),
            preferred_element_type=jnp.float32)
        h = jax.nn.relu(
            jax.lax.dot_general(gath, w1, (((1,), (0,)), ((), ())),
                                preferred_element_type=jnp.float32)
            - cw1 + b1)
        h = jax.nn.relu(
            jax.lax.dot_general(h, w2, (((1,), (0,)), ((), ())),
                                preferred_element_type=jnp.float32)
            + b2)
        pooled = jnp.maximum(pooled, h)
        return pooled

    pooled0 = jnp.full((_TILE, w1.shape[1]), -jnp.inf, dtype=jnp.float32)
    pooled = jax.lax.fori_loop(0, _NSAMPLE, body, pooled0)

    out = jax.lax.dot_general(pooled, ws_ref[...], (((1,), (0,)), ((), ())),
                              preferred_element_type=jnp.float32) + bs_ref[...]
    out_ref[0] = out


def kernel(xyz, features, W1, b1, W2, b2, Ws, bs, npoint, nsample):
    del npoint, nsample
    B, N, _ = xyz.shape
    C = features.shape[-1]
    C1 = W1.shape[1]
    C2 = W2.shape[1]

    xt = jnp.transpose(xyz, (2, 0, 1))
    xtp = jnp.reshape(
        jnp.transpose(jnp.reshape(xt, (3, B, 2, N // 2)), (0, 2, 1, 3)),
        (3, 2 * B, N // 2))

    nx3 = pl.pallas_call(
        _fps_body,
        out_shape=jax.ShapeDtypeStruct((3, B, _NPOINT), jnp.float32),
    )(xtp)
    new_xyz = jnp.transpose(nx3, (1, 2, 0))

    tbl = jnp.concatenate(
        [jnp.transpose(xyz, (0, 2, 1)), jnp.transpose(features, (0, 2, 1))],
        axis=1)

    grid = (B, _NPOINT // _TILE)
    pooled = pl.pallas_call(
        _group_mlp_body,
        grid=grid,
        in_specs=[
            pl.BlockSpec((1, 3 + C, N), lambda b, t: (b, 0, 0)),
            pl.BlockSpec((1, _TILE, 3), lambda b, t: (b, t, 0)),
            pl.BlockSpec((3 + C, C1), lambda b, t: (0, 0)),
            pl.BlockSpec((1, C1), lambda b, t: (0, 0)),
            pl.BlockSpec((C1, C2), lambda b, t: (0, 0)),
            pl.BlockSpec((1, C2), lambda b, t: (0, 0)),
            pl.BlockSpec((C2, C2), lambda b, t: (0, 0)),
            pl.BlockSpec((1, C2), lambda b, t: (0, 0)),
        ],
        out_specs=pl.BlockSpec((1, _TILE, C2), lambda b, t: (b, t, 0)),
        out_shape=jax.ShapeDtypeStruct((B, _NPOINT, C2), jnp.float32),
        scratch_shapes=[pltpu.VMEM((_TILE, N), jnp.float32)],
        compiler_params=pltpu.CompilerParams(
            dimension_semantics=("parallel", "parallel")),
    )(tbl, new_xyz, W1, b1.reshape(1, C1), W2, b2.reshape(1, C2),
      Ws, bs.reshape(1, C2))

    new_features = jnp.transpose(pooled, (0, 2, 1))
    return (new_xyz, new_features)


# speedup vs baseline: 1.1487x; 1.1487x over previous
import functools

import jax
import jax.numpy as jnp
from jax.experimental import pallas as pl
from jax.experimental.pallas import tpu as pltpu

_NPOINT = 1024
_NSAMPLE = 32
_TILE = 256


def _fps_body(xt_ref, nx_ref):
    _, B, N = xt_ref.shape
    xs = xt_ref[0]
    ys = xt_ref[1]
    zs = xt_ref[2]
    lanes = jax.lax.broadcasted_iota(jnp.int32, (B, N), 1).astype(jnp.float32)
    cols = jax.lax.broadcasted_iota(jnp.int32, (B, _NPOINT), 1)

    def body(i, carry):
        dists, far, nxx, nxy, nxz = carry
        mask = lanes == far
        cx = jnp.sum(jnp.where(mask, xs, 0.0), axis=1, keepdims=True)
        cy = jnp.sum(jnp.where(mask, ys, 0.0), axis=1, keepdims=True)
        cz = jnp.sum(jnp.where(mask, zs, 0.0), axis=1, keepdims=True)
        here = cols == i
        nxx = jnp.where(here, cx, nxx)
        nxy = jnp.where(here, cy, nxy)
        nxz = jnp.where(here, cz, nxz)
        dx = xs - cx
        dy = ys - cy
        dz = zs - cz
        d = (dx * dx + dy * dy) + dz * dz
        dists = jnp.minimum(dists, d)
        m = jnp.max(dists, axis=1, keepdims=True)
        far = jnp.min(jnp.where(dists == m, lanes, jnp.float32(N)),
                      axis=1, keepdims=True)
        return dists, far, nxx, nxy, nxz

    dists0 = jnp.full((B, N), 1e10, dtype=jnp.float32)
    far0 = jnp.zeros((B, 1), dtype=jnp.float32)
    nx0 = jnp.zeros((B, _NPOINT), dtype=jnp.float32)
    _, _, nxx, nxy, nxz = jax.lax.fori_loop(
        0, _NPOINT, body, (dists0, far0, nx0, nx0, nx0))
    nx_ref[0] = nxx
    nx_ref[1] = nxy
    nx_ref[2] = nxz


def _group_mlp_body(tbl_ref, nx_ref, w1_ref, b1_ref, w2_ref, b2_ref,
                    ws_ref, bs_ref, out_ref, d2_ref):
    _, CIN, N = tbl_ref.shape
    tbl = tbl_ref[0]
    xt = tbl[0:3, :]
    c = nx_ref[0]

    xsq = xt[0:1] ** 2 + xt[1:2] ** 2 + xt[2:3] ** 2
    csq = jnp.sum(c * c, axis=1, keepdims=True)
    cross = jax.lax.dot_general(c, xt, (((1,), (0,)), ((), ())),
                                preferred_element_type=jnp.float32)
    d2_ref[...] = csq + xsq - 2.0 * cross

    w1 = w1_ref[...]
    cw1 = jax.lax.dot_general(c, w1[0:3, :], (((1,), (0,)), ((), ())),
                              preferred_element_type=jnp.float32)
    b1 = b1_ref[...]
    w2 = w2_ref[...]
    b2 = b2_ref[...]
    lanes = jax.lax.broadcasted_iota(jnp.int32, (_TILE, N), 1).astype(jnp.float32)
    big = jnp.float32(3.0e38)

    def body(j, pooled):
        d2c = d2_ref[...]
        m = jnp.min(d2c, axis=1, keepdims=True)
        idx = jnp.min(jnp.where(d2c == m, lanes, jnp.float32(N)),
                      axis=1, keepdims=True)
        sel = lanes == idx
        d2_ref[...] = jnp.where(sel, big, d2c)
        onehot = sel.astype(jnp.float32)
        gath = jax.lax.dot_general(
            onehot, tbl, (((1,), (1,)), ((), ())),
            preferred_element_type=jnp.float32)
        h = jax.nn.relu(
            jax.lax.dot_general(gath, w1, (((1,), (0,)), ((), ())),
                                preferred_element_type=jnp.float32)
            - cw1 + b1)
        h = jax.nn.relu(
            jax.lax.dot_general(h, w2, (((1,), (0,)), ((), ())),
                                preferred_element_type=jnp.float32)
            + b2)
        pooled = jnp.maximum(pooled, h)
        return pooled

    pooled0 = jnp.full((_TILE, w1.shape[1]), -jnp.inf, dtype=jnp.float32)
    pooled = jax.lax.fori_loop(0, _NSAMPLE, body, pooled0)

    out = jax.lax.dot_general(pooled, ws_ref[...], (((1,), (0,)), ((), ())),
                              preferred_element_type=jnp.float32) + bs_ref[...]
    out_ref[0] = out


def kernel(xyz, features, W1, b1, W2, b2, Ws, bs, npoint, nsample):
    del npoint, nsample
    B, N, _ = xyz.shape
    C = features.shape[-1]
    C1 = W1.shape[1]
    C2 = W2.shape[1]

    xt = jnp.transpose(xyz, (2, 0, 1))

    nx3 = pl.pallas_call(
        _fps_body,
        out_shape=jax.ShapeDtypeStruct((3, B, _NPOINT), jnp.float32),
    )(xt)
    new_xyz = jnp.transpose(nx3, (1, 2, 0))

    tbl = jnp.concatenate(
        [jnp.transpose(xyz, (0, 2, 1)), jnp.transpose(features, (0, 2, 1))],
        axis=1)

    grid = (B, _NPOINT // _TILE)
    pooled = pl.pallas_call(
        _group_mlp_body,
        grid=grid,
        in_specs=[
            pl.BlockSpec((1, 3 + C, N), lambda b, t: (b, 0, 0)),
            pl.BlockSpec((1, _TILE, 3), lambda b, t: (b, t, 0)),
            pl.BlockSpec((3 + C, C1), lambda b, t: (0, 0)),
            pl.BlockSpec((1, C1), lambda b, t: (0, 0)),
            pl.BlockSpec((C1, C2), lambda b, t: (0, 0)),
            pl.BlockSpec((1, C2), lambda b, t: (0, 0)),
            pl.BlockSpec((C2, C2), lambda b, t: (0, 0)),
            pl.BlockSpec((1, C2), lambda b, t: (0, 0)),
        ],
        out_specs=pl.BlockSpec((1, _TILE, C2), lambda b, t: (b, t, 0)),
        out_shape=jax.ShapeDtypeStruct((B, _NPOINT, C2), jnp.float32),
        scratch_shapes=[pltpu.VMEM((_TILE, N), jnp.float32)],
        compiler_params=pltpu.CompilerParams(
            dimension_semantics=("parallel", "parallel")),
    )(tbl, new_xyz, W1, b1.reshape(1, C1), W2, b2.reshape(1, C2),
      Ws, bs.reshape(1, C2))

    new_features = jnp.transpose(pooled, (0, 2, 1))
    return (new_xyz, new_features)


# speedup vs baseline: 1.2149x; 1.0576x over previous
import functools

import jax
import jax.numpy as jnp
from jax.experimental import pallas as pl
from jax.experimental.pallas import tpu as pltpu

_NPOINT = 1024
_NSAMPLE = 32
_TILE = 512


def _fps_body(xt_ref, nx_ref):
    _, B, N = xt_ref.shape
    xs = xt_ref[0]
    ys = xt_ref[1]
    zs = xt_ref[2]
    lanes = jax.lax.broadcasted_iota(jnp.int32, (B, N), 1).astype(jnp.float32)
    cols = jax.lax.broadcasted_iota(jnp.int32, (B, _NPOINT), 1)

    def body(i, carry):
        dists, far, nxx, nxy, nxz = carry
        mask = lanes == far
        cx = jnp.sum(jnp.where(mask, xs, 0.0), axis=1, keepdims=True)
        cy = jnp.sum(jnp.where(mask, ys, 0.0), axis=1, keepdims=True)
        cz = jnp.sum(jnp.where(mask, zs, 0.0), axis=1, keepdims=True)
        here = cols == i
        nxx = jnp.where(here, cx, nxx)
        nxy = jnp.where(here, cy, nxy)
        nxz = jnp.where(here, cz, nxz)
        dx = xs - cx
        dy = ys - cy
        dz = zs - cz
        d = (dx * dx + dy * dy) + dz * dz
        dists = jnp.minimum(dists, d)
        m = jnp.max(dists, axis=1, keepdims=True)
        far = jnp.min(jnp.where(dists == m, lanes, jnp.float32(N)),
                      axis=1, keepdims=True)
        return dists, far, nxx, nxy, nxz

    dists0 = jnp.full((B, N), 1e10, dtype=jnp.float32)
    far0 = jnp.zeros((B, 1), dtype=jnp.float32)
    nx0 = jnp.zeros((B, _NPOINT), dtype=jnp.float32)
    _, _, nxx, nxy, nxz = jax.lax.fori_loop(
        0, _NPOINT, body, (dists0, far0, nx0, nx0, nx0))
    nx_ref[0] = nxx
    nx_ref[1] = nxy
    nx_ref[2] = nxz


def _group_mlp_body(tbl_ref, nx_ref, w1_ref, b1_ref, w2_ref, b2_ref,
                    ws_ref, bs_ref, out_ref, d2_ref):
    _, CIN, N = tbl_ref.shape
    tbl = tbl_ref[0]
    xt = tbl[0:3, :]
    c = nx_ref[0]

    xsq = xt[0:1] ** 2 + xt[1:2] ** 2 + xt[2:3] ** 2
    csq = jnp.sum(c * c, axis=1, keepdims=True)
    cross = jax.lax.dot_general(c, xt, (((1,), (0,)), ((), ())),
                                preferred_element_type=jnp.float32)
    d2_ref[...] = csq + xsq - 2.0 * cross

    w1 = w1_ref[...]
    cw1 = jax.lax.dot_general(c, w1[0:3, :], (((1,), (0,)), ((), ())),
                              preferred_element_type=jnp.float32)
    b1 = b1_ref[...]
    w2 = w2_ref[...]
    b2 = b2_ref[...]
    lanes = jax.lax.broadcasted_iota(jnp.int32, (_TILE, N), 1).astype(jnp.float32)
    big = jnp.float32(3.0e38)

    def body(j, pooled):
        d2c = d2_ref[...]
        m = jnp.min(d2c, axis=1, keepdims=True)
        idx = jnp.min(jnp.where(d2c == m, lanes, jnp.float32(N)),
                      axis=1, keepdims=True)
        sel = lanes == idx
        d2_ref[...] = jnp.where(sel, big, d2c)
        onehot = sel.astype(jnp.float32)
        gath = jax.lax.dot_general(
            onehot, tbl, (((1,), (1,)), ((), ())),
            preferred_element_type=jnp.float32)
        h = jax.nn.relu(
            jax.lax.dot_general(gath, w1, (((1,), (0,)), ((), ())),
                                preferred_element_type=jnp.float32)
            - cw1 + b1)
        h = jax.nn.relu(
            jax.lax.dot_general(h, w2, (((1,), (0,)), ((), ())),
                                preferred_element_type=jnp.float32)
            + b2)
        pooled = jnp.maximum(pooled, h)
        return pooled

    pooled0 = jnp.full((_TILE, w1.shape[1]), -jnp.inf, dtype=jnp.float32)
    pooled = jax.lax.fori_loop(0, _NSAMPLE, body, pooled0)

    out = jax.lax.dot_general(pooled, ws_ref[...], (((1,), (0,)), ((), ())),
                              preferred_element_type=jnp.float32) + bs_ref[...]
    out_ref[0] = out


def kernel(xyz, features, W1, b1, W2, b2, Ws, bs, npoint, nsample):
    del npoint, nsample
    B, N, _ = xyz.shape
    C = features.shape[-1]
    C1 = W1.shape[1]
    C2 = W2.shape[1]

    xt = jnp.transpose(xyz, (2, 0, 1))

    nx3 = pl.pallas_call(
        _fps_body,
        out_shape=jax.ShapeDtypeStruct((3, B, _NPOINT), jnp.float32),
    )(xt)
    new_xyz = jnp.transpose(nx3, (1, 2, 0))

    tbl = jnp.concatenate(
        [jnp.transpose(xyz, (0, 2, 1)), jnp.transpose(features, (0, 2, 1))],
        axis=1)

    grid = (B, _NPOINT // _TILE)
    pooled = pl.pallas_call(
        _group_mlp_body,
        grid=grid,
        in_specs=[
            pl.BlockSpec((1, 3 + C, N), lambda b, t: (b, 0, 0)),
            pl.BlockSpec((1, _TILE, 3), lambda b, t: (b, t, 0)),
            pl.BlockSpec((3 + C, C1), lambda b, t: (0, 0)),
            pl.BlockSpec((1, C1), lambda b, t: (0, 0)),
            pl.BlockSpec((C1, C2), lambda b, t: (0, 0)),
            pl.BlockSpec((1, C2), lambda b, t: (0, 0)),
            pl.BlockSpec((C2, C2), lambda b, t: (0, 0)),
            pl.BlockSpec((1, C2), lambda b, t: (0, 0)),
        ],
        out_specs=pl.BlockSpec((1, _TILE, C2), lambda b, t: (b, t, 0)),
        out_shape=jax.ShapeDtypeStruct((B, _NPOINT, C2), jnp.float32),
        scratch_shapes=[pltpu.VMEM((_TILE, N), jnp.float32)],
        compiler_params=pltpu.CompilerParams(
            dimension_semantics=("parallel", "parallel")),
    )(tbl, new_xyz, W1, b1.reshape(1, C1), W2, b2.reshape(1, C2),
      Ws, bs.reshape(1, C2))

    new_features = jnp.transpose(pooled, (0, 2, 1))
    return (new_xyz, new_features)


# speedup vs baseline: 1.2583x; 1.0357x over previous
import functools

import jax
import jax.numpy as jnp
from jax.experimental import pallas as pl
from jax.experimental.pallas import tpu as pltpu

_NPOINT = 1024
_NSAMPLE = 32
_TILE = 1024


def _fps_body(xt_ref, nx_ref):
    _, B, N = xt_ref.shape
    xs = xt_ref[0]
    ys = xt_ref[1]
    zs = xt_ref[2]
    lanes = jax.lax.broadcasted_iota(jnp.int32, (B, N), 1).astype(jnp.float32)
    cols = jax.lax.broadcasted_iota(jnp.int32, (B, _NPOINT), 1)

    def body(i, carry):
        dists, far, nxx, nxy, nxz = carry
        mask = lanes == far
        cx = jnp.sum(jnp.where(mask, xs, 0.0), axis=1, keepdims=True)
        cy = jnp.sum(jnp.where(mask, ys, 0.0), axis=1, keepdims=True)
        cz = jnp.sum(jnp.where(mask, zs, 0.0), axis=1, keepdims=True)
        here = cols == i
        nxx = jnp.where(here, cx, nxx)
        nxy = jnp.where(here, cy, nxy)
        nxz = jnp.where(here, cz, nxz)
        dx = xs - cx
        dy = ys - cy
        dz = zs - cz
        d = (dx * dx + dy * dy) + dz * dz
        dists = jnp.minimum(dists, d)
        m = jnp.max(dists, axis=1, keepdims=True)
        far = jnp.min(jnp.where(dists == m, lanes, jnp.float32(N)),
                      axis=1, keepdims=True)
        return dists, far, nxx, nxy, nxz

    dists0 = jnp.full((B, N), 1e10, dtype=jnp.float32)
    far0 = jnp.zeros((B, 1), dtype=jnp.float32)
    nx0 = jnp.zeros((B, _NPOINT), dtype=jnp.float32)
    _, _, nxx, nxy, nxz = jax.lax.fori_loop(
        0, _NPOINT, body, (dists0, far0, nx0, nx0, nx0))
    nx_ref[0] = nxx
    nx_ref[1] = nxy
    nx_ref[2] = nxz


def _group_mlp_body(tbl_ref, nx_ref, w1_ref, b1_ref, w2_ref, b2_ref,
                    ws_ref, bs_ref, out_ref, d2_ref):
    _, CIN, N = tbl_ref.shape
    tbl = tbl_ref[0]
    xt = tbl[0:3, :]
    c = nx_ref[0]

    xsq = xt[0:1] ** 2 + xt[1:2] ** 2 + xt[2:3] ** 2
    csq = jnp.sum(c * c, axis=1, keepdims=True)
    cross = jax.lax.dot_general(c, xt, (((1,), (0,)), ((), ())),
                                preferred_element_type=jnp.float32)
    d2_ref[...] = csq + xsq - 2.0 * cross

    w1 = w1_ref[...]
    cw1 = jax.lax.dot_general(c, w1[0:3, :], (((1,), (0,)), ((), ())),
                              preferred_element_type=jnp.float32)
    b1 = b1_ref[...]
    w2 = w2_ref[...]
    b2 = b2_ref[...]
    lanes = jax.lax.broadcasted_iota(jnp.int32, (_TILE, N), 1).astype(jnp.float32)
    big = jnp.float32(3.0e38)

    def body(j, pooled):
        d2c = d2_ref[...]
        m = jnp.min(d2c, axis=1, keepdims=True)
        idx = jnp.min(jnp.where(d2c == m, lanes, jnp.float32(N)),
                      axis=1, keepdims=True)
        sel = lanes == idx
        d2_ref[...] = jnp.where(sel, big, d2c)
        onehot = sel.astype(jnp.float32)
        gath = jax.lax.dot_general(
            onehot, tbl, (((1,), (1,)), ((), ())),
            preferred_element_type=jnp.float32)
        h = jax.nn.relu(
            jax.lax.dot_general(gath, w1, (((1,), (0,)), ((), ())),
                                preferred_element_type=jnp.float32)
            - cw1 + b1)
        h = jax.nn.relu(
            jax.lax.dot_general(h, w2, (((1,), (0,)), ((), ())),
                                preferred_element_type=jnp.float32)
            + b2)
        pooled = jnp.maximum(pooled, h)
        return pooled

    pooled0 = jnp.full((_TILE, w1.shape[1]), -jnp.inf, dtype=jnp.float32)
    pooled = jax.lax.fori_loop(0, _NSAMPLE, body, pooled0)

    out = jax.lax.dot_general(pooled, ws_ref[...], (((1,), (0,)), ((), ())),
                              preferred_element_type=jnp.float32) + bs_ref[...]
    out_ref[0] = out


def kernel(xyz, features, W1, b1, W2, b2, Ws, bs, npoint, nsample):
    del npoint, nsample
    B, N, _ = xyz.shape
    C = features.shape[-1]
    C1 = W1.shape[1]
    C2 = W2.shape[1]

    xt = jnp.transpose(xyz, (2, 0, 1))

    nx3 = pl.pallas_call(
        _fps_body,
        out_shape=jax.ShapeDtypeStruct((3, B, _NPOINT), jnp.float32),
    )(xt)
    new_xyz = jnp.transpose(nx3, (1, 2, 0))

    tbl = jnp.concatenate(
        [jnp.transpose(xyz, (0, 2, 1)), jnp.transpose(features, (0, 2, 1))],
        axis=1)

    grid = (B, _NPOINT // _TILE)
    pooled = pl.pallas_call(
        _group_mlp_body,
        grid=grid,
        in_specs=[
            pl.BlockSpec((1, 3 + C, N), lambda b, t: (b, 0, 0)),
            pl.BlockSpec((1, _TILE, 3), lambda b, t: (b, t, 0)),
            pl.BlockSpec((3 + C, C1), lambda b, t: (0, 0)),
            pl.BlockSpec((1, C1), lambda b, t: (0, 0)),
            pl.BlockSpec((C1, C2), lambda b, t: (0, 0)),
            pl.BlockSpec((1, C2), lambda b, t: (0, 0)),
            pl.BlockSpec((C2, C2), lambda b, t: (0, 0)),
            pl.BlockSpec((1, C2), lambda b, t: (0, 0)),
        ],
        out_specs=pl.BlockSpec((1, _TILE, C2), lambda b, t: (b, t, 0)),
        out_shape=jax.ShapeDtypeStruct((B, _NPOINT, C2), jnp.float32),
        scratch_shapes=[pltpu.VMEM((_TILE, N), jnp.float32)],
        compiler_params=pltpu.CompilerParams(
            dimension_semantics=("parallel", "parallel")),
    )(tbl, new_xyz, W1, b1.reshape(1, C1), W2, b2.reshape(1, C2),
      Ws, bs.reshape(1, C2))

    new_features = jnp.transpose(pooled, (0, 2, 1))
    return (new_xyz, new_features)
